# baseline (speedup 1.0000x reference)
import functools

import jax
import jax.numpy as jnp
from jax.experimental import pallas as pl
from jax.experimental.pallas import tpu as pltpu
from jax.experimental.pallas import tpu_sc as plsc

_WINDOW = 128
_BLK = 2048


def _sc_gather(user_table, movie_table, user_idx, movie_idx):
    batch = user_idx.shape[1]
    emb = user_table.shape[1]
    mesh = plsc.VectorSubcoreMesh(core_axis_name="core", subcore_axis_name="subcore")

    @functools.partial(
        pl.kernel,
        out_type=(
            jax.ShapeDtypeStruct((batch, emb), user_table.dtype),
            jax.ShapeDtypeStruct((batch, emb), movie_table.dtype),
        ),
        mesh=mesh,
        compiler_params=pltpu.CompilerParams(use_tc_tiling_on_sc=False),
    )
    def gather_kernel(ut_hbm, mt_hbm, ui_hbm, mi_hbm, uo_hbm, mo_hbm):
        def body(ui_v, mi_v, uo_v, mo_v):
            pltpu.sync_copy(ut_hbm.at[ui_v.at[0]], uo_v)
            pltpu.sync_copy(mt_hbm.at[mi_v.at[0]], mo_v)

        pltpu.emit_pipeline(
            body,
            grid=(batch // _WINDOW,),
            in_specs=[
                pl.BlockSpec((1, _WINDOW), lambda i: (0, i)),
                pl.BlockSpec((1, _WINDOW), lambda i: (0, i)),
            ],
            out_specs=[
                pl.BlockSpec((_WINDOW, emb), lambda i: (i, 0)),
                pl.BlockSpec((_WINDOW, emb), lambda i: (i, 0)),
            ],
            core_axis_name=("core", "subcore"),
            dimension_semantics=(pltpu.PARALLEL,),
        )(ui_hbm, mi_hbm, uo_hbm, mo_hbm)

    return gather_kernel(user_table, movie_table, user_idx, movie_idx)


def _mlp_body(ue, me, g, w1, b1, w2, b2, out):
    emb = ue.shape[1]
    ng = g.shape[1]
    h = jnp.dot(ue[...], w1[0:emb, :], preferred_element_type=jnp.float32)
    h += jnp.dot(me[...], w1[emb:2 * emb, :], preferred_element_type=jnp.float32)
    h += jnp.dot(g[...], w1[2 * emb:2 * emb + ng, :], preferred_element_type=jnp.float32)
    h = jnp.maximum(h + b1[...], 0.0)
    out[...] = jnp.dot(h, w2[...], preferred_element_type=jnp.float32) + b2[...]


def _tc_mlp(ue, me, genre, w1, b1, w2, b2):
    batch, emb = ue.shape
    ng = genre.shape[1]
    hidden = w1.shape[1]
    grid = (batch // _BLK,)
    return pl.pallas_call(
        _mlp_body,
        grid=grid,
        in_specs=[
            pl.BlockSpec((_BLK, emb), lambda i: (i, 0)),
            pl.BlockSpec((_BLK, emb), lambda i: (i, 0)),
            pl.BlockSpec((_BLK, ng), lambda i: (i, 0)),
            pl.BlockSpec((2 * emb + ng, hidden), lambda i: (0, 0)),
            pl.BlockSpec((1, hidden), lambda i: (0, 0)),
            pl.BlockSpec((hidden, 1), lambda i: (0, 0)),
            pl.BlockSpec((1, 1), lambda i: (0, 0)),
        ],
        out_specs=pl.BlockSpec((_BLK, 1), lambda i: (i, 0)),
        out_shape=jax.ShapeDtypeStruct((batch, 1), jnp.float32),
    )(ue, me, genre, w1, b1, w2, b2)


def kernel(user, movie, genre_vec, user_table, movie_table, W1, b1, W2, b2):
    batch = user.shape[0]
    ue, me = _sc_gather(
        user_table,
        movie_table,
        user.reshape(1, batch),
        movie.reshape(1, batch),
    )
    out = _tc_mlp(
        ue, me, genre_vec, W1,
        b1.reshape(1, -1), W2, b2.reshape(1, 1),
    )
    return out.reshape(batch)


# speedup vs baseline: 1.6425x; 1.6425x over previous
import functools

import jax
import jax.numpy as jnp
from jax.experimental import pallas as pl
from jax.experimental.pallas import tpu as pltpu
from jax.experimental.pallas import tpu_sc as plsc

_WINDOW = 128
_BLK = 2048


def _sc_gather(user_table, movie_table, user_idx, movie_idx):
    batch = user_idx.shape[0]
    emb = user_table.shape[1]
    mesh = plsc.VectorSubcoreMesh(core_axis_name="core", subcore_axis_name="subcore")
    nw = mesh.num_cores * mesh.num_subcores
    per_w = batch // nw

    @functools.partial(
        pl.kernel,
        out_type=(
            jax.ShapeDtypeStruct((batch, emb), user_table.dtype),
            jax.ShapeDtypeStruct((batch, emb), movie_table.dtype),
        ),
        mesh=mesh,
        scratch_types=[
            pltpu.VMEM((per_w,), jnp.int32),
            pltpu.VMEM((per_w,), jnp.int32),
            pltpu.VMEM((per_w // 2, emb), jnp.float32),
            pltpu.VMEM((per_w // 2, emb), jnp.float32),
            pltpu.SemaphoreType.DMA,
            pltpu.SemaphoreType.DMA,
        ],
    )
    def gather_kernel(ut_hbm, mt_hbm, ui_hbm, mi_hbm, uo_hbm, mo_hbm,
                      ui_v, mi_v, ur_v, mr_v, sem_u, sem_m):
        wid = jax.lax.axis_index("subcore") * mesh.num_cores + jax.lax.axis_index("core")
        base = wid * per_w
        half = per_w // 2
        pltpu.sync_copy(ui_hbm.at[pl.ds(base, per_w)], ui_v)
        pltpu.sync_copy(mi_hbm.at[pl.ds(base, per_w)], mi_v)

        for c in range(2):
            off = c * half

            @pl.loop(0, half, step=16)
            def _(j):
                uvec = ui_v[pl.ds(off + j, 16)]
                mvec = mi_v[pl.ds(off + j, 16)]
                for k in range(16):
                    pltpu.async_copy(ut_hbm.at[uvec[k]], ur_v.at[j + k], sem_u)
                    pltpu.async_copy(mt_hbm.at[mvec[k]], mr_v.at[j + k], sem_m)

            pltpu.make_async_copy(ut_hbm.at[pl.ds(0, half)], ur_v, sem_u).wait()
            pltpu.make_async_copy(mt_hbm.at[pl.ds(0, half)], mr_v, sem_m).wait()
            pltpu.sync_copy(ur_v, uo_hbm.at[pl.ds(base + off, half)])
            pltpu.sync_copy(mr_v, mo_hbm.at[pl.ds(base + off, half)])

    return gather_kernel(user_table, movie_table, user_idx, movie_idx)


def _mlp_body(ue, me, g, w1, b1, w2, b2, out):
    emb = ue.shape[1]
    ng = g.shape[1]
    h = jnp.dot(ue[...], w1[0:emb, :], preferred_element_type=jnp.float32)
    h += jnp.dot(me[...], w1[emb:2 * emb, :], preferred_element_type=jnp.float32)
    h += jnp.dot(g[...], w1[2 * emb:2 * emb + ng, :], preferred_element_type=jnp.float32)
    h = jnp.maximum(h + b1[...], 0.0)
    out[...] = jnp.dot(h, w2[...], preferred_element_type=jnp.float32) + b2[...]


def _tc_mlp(ue, me, genre, w1, b1, w2, b2):
    batch, emb = ue.shape
    ng = genre.shape[1]
    hidden = w1.shape[1]
    grid = (batch // _BLK,)
    return pl.pallas_call(
        _mlp_body,
        grid=grid,
        in_specs=[
            pl.BlockSpec((_BLK, emb), lambda i: (i, 0)),
            pl.BlockSpec((_BLK, emb), lambda i: (i, 0)),
            pl.BlockSpec((_BLK, ng), lambda i: (i, 0)),
            pl.BlockSpec((2 * emb + ng, hidden), lambda i: (0, 0)),
            pl.BlockSpec((1, hidden), lambda i: (0, 0)),
            pl.BlockSpec((hidden, 1), lambda i: (0, 0)),
            pl.BlockSpec((1, 1), lambda i: (0, 0)),
        ],
        out_specs=pl.BlockSpec((_BLK, 1), lambda i: (i, 0)),
        out_shape=jax.ShapeDtypeStruct((batch, 1), jnp.float32),
    )(ue, me, genre, w1, b1, w2, b2)


def kernel(user, movie, genre_vec, user_table, movie_table, W1, b1, W2, b2):
    batch = user.shape[0]
    ue, me = _sc_gather(user_table, movie_table, user, movie)
    out = _tc_mlp(
        ue, me, genre_vec, W1,
        b1.reshape(1, -1), W2, b2.reshape(1, 1),
    )
    return out.reshape(batch)
